# Initial kernel scaffold; baseline (speedup 1.0000x reference)
import jax
import jax.numpy as jnp
from jax.experimental import pallas as pl


def kernel(x, w_dw, g1, b1, w_pw, g2, b2):
    raise NotImplementedError("write your pallas kernel here")


# speedup vs baseline: 1.0531x; 1.0531x over previous
import functools

import jax
import jax.numpy as jnp
from jax.experimental import pallas as pl
from jax.experimental.pallas import tpu as pltpu

_EPS = 1e-5
_VMEM_LIMIT = 64 * 1024 * 1024


def _conv3x3(x_ref, w_ref, Ho, Wo):
    w9 = w_ref[...].astype(jnp.float32)
    acc = jnp.zeros((Ho, Wo, x_ref.shape[-1]), jnp.float32)
    for k in range(9):
        di, dj = divmod(k, 3)
        acc = acc + x_ref[di:di + Ho, dj:dj + Wo, :].astype(jnp.float32) * w9[k]
    return acc.reshape(Ho * Wo, x_ref.shape[-1])


def _k1_stats(x_ref, w_ref, stats_ref, *, Ho, Wo):
    y = _conv3x3(x_ref, w_ref, Ho, Wo)
    stats_ref[0:1, :] = jnp.sum(y, axis=0, keepdims=True)
    stats_ref[1:2, :] = jnp.sum(y * y, axis=0, keepdims=True)


def _k2_gram(x_ref, w_ref, sc1_ref, sh1_ref, suma_ref, gram_ref, *, Ho, Wo):
    y = _conv3x3(x_ref, w_ref, Ho, Wo)
    a = jnp.maximum(y * sc1_ref[...] + sh1_ref[...], 0.0)
    suma_ref[...] = jnp.sum(a, axis=0, keepdims=True)
    ab = a.astype(jnp.bfloat16)
    gram_ref[...] = jax.lax.dot_general(
        ab, ab, (((0,), (0,)), ((), ())),
        preferred_element_type=jnp.float32)


def _k3_out(x_ref, w_ref, sc1_ref, sh1_ref, wps_ref, sh2_ref, out_ref, *, Ho, Wo):
    y = _conv3x3(x_ref, w_ref, Ho, Wo)
    a = jnp.maximum(y * sc1_ref[...] + sh1_ref[...], 0.0)
    ab = a.astype(jnp.bfloat16)
    zt = jax.lax.dot_general(
        wps_ref[...], ab, (((0,), (1,)), ((), ())),
        preferred_element_type=jnp.float32)
    out_ref[...] = jnp.maximum(zt + sh2_ref[...], 0.0)


def _fold(sum_, sumsq, gamma, beta, inv_cnt):
    mean = sum_ * inv_cnt
    var = jnp.maximum(sumsq * inv_cnt - mean * mean, 0.0)
    scale = gamma * jax.lax.rsqrt(var + _EPS)
    return scale, beta - mean * scale


@jax.jit
def kernel(x, w_dw, g1, b1, w_pw, g2, b2):
    N, C, H, W = x.shape
    Co = w_pw.shape[0]
    Ho, Wo = H, W
    Hp, Wp = H + 2, W + 2
    S = Ho * Wo
    inv_cnt = 1.0 / float(N * S)

    x_pad = jnp.pad(jnp.transpose(x, (0, 2, 3, 1)),
                    ((0, 0), (1, 1), (1, 1), (0, 0))).astype(jnp.bfloat16)
    wdw = jnp.transpose(w_dw.reshape(C, 9), (1, 0))
    wpw = jnp.transpose(w_pw.reshape(Co, C), (1, 0))

    img = pl.BlockSpec((None, Hp, Wp, C), lambda n: (n, 0, 0, 0))
    cst = lambda shape: pl.BlockSpec(shape, lambda n: (0,) * len(shape))
    par = pltpu.CompilerParams(dimension_semantics=("parallel",),
                               vmem_limit_bytes=_VMEM_LIMIT)

    stats1 = pl.pallas_call(
        functools.partial(_k1_stats, Ho=Ho, Wo=Wo),
        out_shape=jax.ShapeDtypeStruct((N, 2, C), jnp.float32),
        grid=(N,),
        in_specs=[img, cst((9, C))],
        out_specs=pl.BlockSpec((None, 2, C), lambda n: (n, 0, 0)),
        compiler_params=par,
    )(x_pad, wdw)
    scale1, shift1 = _fold(jnp.sum(stats1[:, 0, :], axis=0),
                           jnp.sum(stats1[:, 1, :], axis=0), g1, b1, inv_cnt)

    suma, gram = pl.pallas_call(
        functools.partial(_k2_gram, Ho=Ho, Wo=Wo),
        out_shape=(jax.ShapeDtypeStruct((N, 1, C), jnp.float32),
                   jax.ShapeDtypeStruct((N, C, C), jnp.float32)),
        grid=(N,),
        in_specs=[img, cst((9, C)), cst((1, C)), cst((1, C))],
        out_specs=(pl.BlockSpec((None, 1, C), lambda n: (n, 0, 0)),
                   pl.BlockSpec((None, C, C), lambda n: (n, 0, 0))),
        compiler_params=par,
    )(x_pad, wdw, scale1.reshape(1, C), shift1.reshape(1, C))
    sum_z = jnp.sum(suma, axis=(0, 1)) @ wpw
    gram_t = jnp.sum(gram, axis=0)
    sumsq_z = jnp.sum(wpw * (gram_t @ wpw), axis=0)
    scale2, shift2 = _fold(sum_z, sumsq_z, g2, b2, inv_cnt)

    wps = (wpw * scale2[None, :]).astype(jnp.bfloat16)
    out = pl.pallas_call(
        functools.partial(_k3_out, Ho=Ho, Wo=Wo),
        out_shape=jax.ShapeDtypeStruct((N, Co, S), jnp.float32),
        grid=(N,),
        in_specs=[img, cst((9, C)), cst((1, C)), cst((1, C)),
                  cst((C, Co)), cst((Co, 1))],
        out_specs=pl.BlockSpec((None, Co, S), lambda n: (n, 0, 0)),
        compiler_params=par,
    )(x_pad, wdw, scale1.reshape(1, C), shift1.reshape(1, C),
      wps, shift2.reshape(Co, 1))
    return out.reshape(N, Co, Ho, Wo)
